# baseline (speedup 1.0000x reference)
import functools

import jax
import jax.numpy as jnp
from jax import lax
from jax.experimental import pallas as pl
from jax.experimental.pallas import tpu as pltpu
from jax.experimental.pallas import tpu_sc as plsc

_B, _T = 16384, 2048
_N = _B * _T
_NC, _NS, _L = 2, 16, 16
_NW = _NC * _NS
_PER_W = _N // _NW
_CH = 16384
_NCHUNK = _PER_W // _CH
_GROUPS = _CH // 64


def _take16(v, idx):
    dn = lax.GatherDimensionNumbers(
        offset_dims=(), collapsed_slice_dims=(0,), start_index_map=(0,))
    return lax.gather(v, idx[:, None], dn, slice_sizes=(1,),
                      mode=lax.GatherScatterMode.PROMISE_IN_BOUNDS)


@functools.partial(
    pl.kernel,
    out_type=[
        jax.ShapeDtypeStruct((_NW, _L), jnp.float32),
        jax.ShapeDtypeStruct((_NW, _L), jnp.int32),
    ],
    mesh=plsc.VectorSubcoreMesh(core_axis_name="c", subcore_axis_name="s"),
    scratch_types=[
        pltpu.VMEM((2, _CH), jnp.float32),
        pltpu.VMEM((2, _CH), jnp.float32),
        pltpu.VMEM((2, _CH // 4), jnp.int32),
        pltpu.VMEM((_L,), jnp.float32),
        pltpu.VMEM((_L,), jnp.int32),
        pltpu.SemaphoreType.DMA,
        pltpu.SemaphoreType.DMA,
    ],
)
def _sc_mse(p_hbm, t_hbm, m_hbm, sums_hbm, cnts_hbm,
            pbuf, tbuf, mbuf, svec, cvec, sem0, sem1):
    wid = lax.axis_index("s") * _NC + lax.axis_index("c")
    base = wid * _PER_W

    lanes = lax.iota(jnp.int32, _L)
    word_idx = lanes >> 2
    bit_shift = (lanes & 3) << 3

    def _offs(c):
        eoff = pl.multiple_of(base + c * _CH, _CH)
        moff = pl.multiple_of(wid * (_PER_W // 4) + c * (_CH // 4), _CH // 4)
        return eoff, moff

    sems = (sem0, sem1)

    def _issue(c, b):
        eoff, moff = _offs(c)
        pltpu.async_copy(p_hbm.at[pl.ds(eoff, _CH)], pbuf.at[b], sems[b])
        pltpu.async_copy(t_hbm.at[pl.ds(eoff, _CH)], tbuf.at[b], sems[b])
        pltpu.async_copy(m_hbm.at[pl.ds(moff, _CH // 4)], mbuf.at[b], sems[b])

    def _drain(c, b):
        eoff, moff = _offs(c)
        pltpu.make_async_copy(p_hbm.at[pl.ds(eoff, _CH)], pbuf.at[b],
                              sems[b]).wait()
        pltpu.make_async_copy(t_hbm.at[pl.ds(eoff, _CH)], tbuf.at[b],
                              sems[b]).wait()
        pltpu.make_async_copy(m_hbm.at[pl.ds(moff, _CH // 4)], mbuf.at[b],
                              sems[b]).wait()

    _issue(0, 0)
    _issue(1, 1)

    def chunk_body(c, b, carry):
        _drain(c, b)
        pb, tb, mb = pbuf.at[b], tbuf.at[b], mbuf.at[b]

        def inner(j, carry):
            a0, a1, a2, a3, cnt = carry
            off = j * 64
            mw = mb[pl.ds(j * 16, 16)]
            cnt = cnt + ((mw * 0x01010101) >> 24)

            def grp(jj, acc):
                g = _take16(mw, word_idx + 4 * jj)
                keep = ((g >> bit_shift) & 1) == 0
                d = pb[pl.ds(off + 16 * jj, 16)] - tb[pl.ds(off + 16 * jj, 16)]
                dk = jnp.where(keep, d, 0.0)
                return acc + dk * dk

            return grp(0, a0), grp(1, a1), grp(2, a2), grp(3, a3), cnt

        carry = lax.fori_loop(0, _GROUPS, inner, carry)

        @pl.when(c + 2 < _NCHUNK)
        def _():
            _issue(c + 2, b)

        return carry

    zf = jnp.zeros((_L,), jnp.float32)
    zi = jnp.zeros((_L,), jnp.int32)

    def outer(g, carry):
        carry = chunk_body(2 * g, 0, carry)
        carry = chunk_body(2 * g + 1, 1, carry)
        return carry

    a0, a1, a2, a3, cnt = lax.fori_loop(
        0, _NCHUNK // 2, outer, (zf, zf, zf, zf, zi))

    svec[...] = (a0 + a1) + (a2 + a3)
    cvec[...] = cnt
    pltpu.sync_copy(svec, sums_hbm.at[wid])
    pltpu.sync_copy(cvec, cnts_hbm.at[wid])


def kernel(preds, target, mask):
    mwords = mask.reshape(-1).view(jnp.int32)
    sums, cnts = _sc_mse(preds.reshape(-1), target.reshape(-1), mwords)
    loss = jnp.sum(sums)
    kept = jnp.float32(_N) - jnp.sum(cnts).astype(jnp.float32)
    return loss / kept


# speedup vs baseline: 46.5073x; 46.5073x over previous
import jax
import jax.numpy as jnp
from jax.experimental import pallas as pl

_B, _T = 16384, 2048
_N = _B * _T
_BLK = 256
_GRID = _B // _BLK


def _fold8(x):
    acc = x[0:8, :]
    for k in range(1, _BLK // 8):
        acc = acc + x[8 * k:8 * (k + 1), :]
    return acc


def _mse_block(p_ref, t_ref, m_ref, s_ref, c_ref):
    d = p_ref[...] - t_ref[...]
    mconv = m_ref[...].astype(jnp.float32)
    dm = d * (1.0 - mconv)

    @pl.when(pl.program_id(0) == 0)
    def _():
        s_ref[...] = jnp.zeros_like(s_ref)
        c_ref[...] = jnp.zeros_like(c_ref)

    s_ref[...] += _fold8(dm * dm)
    c_ref[...] += _fold8(mconv)


def kernel(preds, target, mask):
    sums, mcnt = pl.pallas_call(
        _mse_block,
        grid=(_GRID,),
        in_specs=[
            pl.BlockSpec((_BLK, _T), lambda i: (i, 0)),
            pl.BlockSpec((_BLK, _T), lambda i: (i, 0)),
            pl.BlockSpec((_BLK, _T), lambda i: (i, 0)),
        ],
        out_specs=[
            pl.BlockSpec((8, _T), lambda i: (0, 0)),
            pl.BlockSpec((8, _T), lambda i: (0, 0)),
        ],
        out_shape=[
            jax.ShapeDtypeStruct((8, _T), jnp.float32),
            jax.ShapeDtypeStruct((8, _T), jnp.float32),
        ],
    )(preds, target, mask.view(jnp.int8))
    loss = jnp.sum(sums)
    kept = jnp.float32(_N) - jnp.sum(mcnt)
    return loss / kept


# speedup vs baseline: 50.9197x; 1.0949x over previous
import jax
import jax.numpy as jnp
from jax.experimental import pallas as pl

_B, _T = 16384, 2048
_N = _B * _T
_BLK = 512
_GRID = _B // _BLK


def _fold8(x):
    acc = x[0:8, :]
    for k in range(1, _BLK // 8):
        acc = acc + x[8 * k:8 * (k + 1), :]
    return acc


def _mse_block(p_ref, t_ref, m_ref, s_ref, c_ref):
    d = p_ref[...] - t_ref[...]
    mconv = m_ref[...].astype(jnp.float32)
    dm = d * (1.0 - mconv)

    @pl.when(pl.program_id(0) == 0)
    def _():
        s_ref[...] = jnp.zeros_like(s_ref)
        c_ref[...] = jnp.zeros_like(c_ref)

    s_ref[...] += _fold8(dm * dm)
    c_ref[...] += _fold8(mconv)


def kernel(preds, target, mask):
    sums, mcnt = pl.pallas_call(
        _mse_block,
        grid=(_GRID,),
        in_specs=[
            pl.BlockSpec((_BLK, _T), lambda i: (i, 0)),
            pl.BlockSpec((_BLK, _T), lambda i: (i, 0)),
            pl.BlockSpec((_BLK, _T), lambda i: (i, 0)),
        ],
        out_specs=[
            pl.BlockSpec((8, _T), lambda i: (0, 0)),
            pl.BlockSpec((8, _T), lambda i: (0, 0)),
        ],
        out_shape=[
            jax.ShapeDtypeStruct((8, _T), jnp.float32),
            jax.ShapeDtypeStruct((8, _T), jnp.float32),
        ],
    )(preds, target, mask.view(jnp.int8))
    loss = jnp.sum(sums)
    kept = jnp.float32(_N) - jnp.sum(mcnt)
    return loss / kept
